# baseline (device time: 34359 ns/iter reference)
import jax
import jax.numpy as jnp
from jax import lax
from jax.experimental import pallas as pl
from jax.experimental.pallas import tpu as pltpu


def kernel(O, Wo):
    B, S, H, D = O.shape
    K = H * D
    N = Wo.shape[1]
    S_half = S // 2

    O2 = O.reshape(B, S, K).astype(jnp.bfloat16)
    Wo_b = Wo.astype(jnp.bfloat16)

    def body(o_ref, wo_ref, out_ref, send_buf, recv_buf, send_sems, recv_sems):
        my_x = lax.axis_index("x")
        my_y = lax.axis_index("y")
        my_z = lax.axis_index("z")
        partner = (my_x, my_y, 1 - my_z)

        barrier_sem = pltpu.get_barrier_semaphore()
        pl.semaphore_signal(
            barrier_sem, inc=1,
            device_id=partner, device_id_type=pl.DeviceIdType.MESH,
        )
        pl.semaphore_wait(barrier_sem, 1)

        wo = wo_ref[...]
        p0 = (1 - my_z) * S_half
        m0 = my_z * S_half

        rdmas = []
        for b in range(B):
            acc = jnp.dot(
                o_ref[b, pl.ds(p0, S_half), :], wo,
                preferred_element_type=jnp.float32,
            )
            send_buf[b] = acc.astype(jnp.bfloat16)
            r = pltpu.make_async_remote_copy(
                src_ref=send_buf.at[b],
                dst_ref=recv_buf.at[b],
                send_sem=send_sems.at[b],
                recv_sem=recv_sems.at[b],
                device_id=partner,
                device_id_type=pl.DeviceIdType.MESH,
            )
            r.start()
            rdmas.append(r)

        for b in range(B):
            out_ref[b] = jnp.dot(
                o_ref[b, pl.ds(m0, S_half), :], wo,
                preferred_element_type=jnp.float32,
            )

        for b in range(B):
            rdmas[b].wait_recv()
            out_ref[b] = out_ref[b] + recv_buf[b].astype(jnp.float32)
        for b in range(B):
            rdmas[b].wait_send()

    out_shape = jax.ShapeDtypeStruct((B, S_half, N), jnp.float32)
    return pl.pallas_call(
        body,
        out_shape=out_shape,
        in_specs=[
            pl.BlockSpec(memory_space=pltpu.VMEM),
            pl.BlockSpec(memory_space=pltpu.VMEM),
        ],
        out_specs=pl.BlockSpec(memory_space=pltpu.VMEM),
        scratch_shapes=[
            pltpu.VMEM((B, S_half, N), jnp.bfloat16),
            pltpu.VMEM((B, S_half, N), jnp.bfloat16),
            pltpu.SemaphoreType.DMA((B,)),
            pltpu.SemaphoreType.DMA((B,)),
        ],
        compiler_params=pltpu.CompilerParams(collective_id=0),
    )(O2, Wo_b)


# device time: 34240 ns/iter; 1.0035x vs baseline; 1.0035x over previous
import jax
import jax.numpy as jnp
from jax import lax
from jax.experimental import pallas as pl
from jax.experimental.pallas import tpu as pltpu


def kernel(O, Wo):
    B, S, H, D = O.shape
    K = H * D
    N = Wo.shape[1]
    S_half = S // 2

    O2 = O.reshape(B, S * H, D)

    def body(o_ref, wo_ref, out_ref, send_buf, recv_buf, send_sems, recv_sems):
        my_x = lax.axis_index("x")
        my_y = lax.axis_index("y")
        my_z = lax.axis_index("z")
        partner = (my_x, my_y, 1 - my_z)

        barrier_sem = pltpu.get_barrier_semaphore()
        pl.semaphore_signal(
            barrier_sem, inc=1,
            device_id=partner, device_id_type=pl.DeviceIdType.MESH,
        )
        pl.semaphore_wait(barrier_sem, 1)

        wo = wo_ref[...].astype(jnp.bfloat16)
        p0 = (1 - my_z) * S_half
        m0 = my_z * S_half

        rdmas = []
        for b in range(B):
            o3 = jnp.reshape(
                o_ref[b, pl.ds(p0 * H, S_half * H), :], (S_half, H, D)
            ).astype(jnp.bfloat16)
            o_blk = jnp.reshape(o3, (S_half, K))
            acc = jnp.dot(o_blk, wo, preferred_element_type=jnp.float32)
            send_buf[b] = acc.astype(jnp.bfloat16)
            r = pltpu.make_async_remote_copy(
                src_ref=send_buf.at[b],
                dst_ref=recv_buf.at[b],
                send_sem=send_sems.at[b],
                recv_sem=recv_sems.at[b],
                device_id=partner,
                device_id_type=pl.DeviceIdType.MESH,
            )
            r.start()
            rdmas.append(r)

        for b in range(B):
            o3 = jnp.reshape(
                o_ref[b, pl.ds(m0 * H, S_half * H), :], (S_half, H, D)
            ).astype(jnp.bfloat16)
            o_blk = jnp.reshape(o3, (S_half, K))
            out_ref[b] = jnp.dot(o_blk, wo, preferred_element_type=jnp.float32)

        for b in range(B):
            rdmas[b].wait_recv()
            out_ref[b] = out_ref[b] + recv_buf[b].astype(jnp.float32)
        for b in range(B):
            rdmas[b].wait_send()

    out_shape = jax.ShapeDtypeStruct((B, S_half, N), jnp.float32)
    return pl.pallas_call(
        body,
        out_shape=out_shape,
        in_specs=[
            pl.BlockSpec(memory_space=pltpu.VMEM),
            pl.BlockSpec(memory_space=pltpu.VMEM),
        ],
        out_specs=pl.BlockSpec(memory_space=pltpu.VMEM),
        scratch_shapes=[
            pltpu.VMEM((B, S_half, N), jnp.bfloat16),
            pltpu.VMEM((B, S_half, N), jnp.bfloat16),
            pltpu.SemaphoreType.DMA((B,)),
            pltpu.SemaphoreType.DMA((B,)),
        ],
        compiler_params=pltpu.CompilerParams(collective_id=0),
    )(O2, Wo)


# device time: 32742 ns/iter; 1.0494x vs baseline; 1.0458x over previous
import jax
import jax.numpy as jnp
from jax import lax
from jax.experimental import pallas as pl
from jax.experimental.pallas import tpu as pltpu

N_CHUNKS = 8
ROWS = 128


def kernel(O, Wo):
    B, S, H, D = O.shape
    K = H * D
    N = Wo.shape[1]
    S_half = S // 2

    O2 = O.reshape(B, S, K)

    def body(o_ref, wo_ref, out_ref, send_buf, recv_buf, send_sems, recv_sems):
        my_x = lax.axis_index("x")
        my_y = lax.axis_index("y")
        my_z = lax.axis_index("z")
        partner = (my_x, my_y, 1 - my_z)

        barrier_sem = pltpu.get_barrier_semaphore()
        pl.semaphore_signal(
            barrier_sem, inc=1,
            device_id=partner, device_id_type=pl.DeviceIdType.MESH,
        )
        pl.semaphore_wait(barrier_sem, 1)

        wo = wo_ref[...].astype(jnp.bfloat16)
        p0 = (1 - my_z) * S_half
        m0 = my_z * S_half

        rdmas = []
        for c in range(N_CHUNKS):
            b, r = divmod(c, S_half // ROWS)
            acc = jnp.dot(
                o_ref[b, pl.ds(p0 + r * ROWS, ROWS), :].astype(jnp.bfloat16),
                wo,
                preferred_element_type=jnp.float32,
            )
            send_buf[c] = acc.astype(jnp.bfloat16)
            rdma = pltpu.make_async_remote_copy(
                src_ref=send_buf.at[c],
                dst_ref=recv_buf.at[c],
                send_sem=send_sems.at[c],
                recv_sem=recv_sems.at[c],
                device_id=partner,
                device_id_type=pl.DeviceIdType.MESH,
            )
            rdma.start()
            rdmas.append(rdma)

        for b in range(B):
            out_ref[b] = jnp.dot(
                o_ref[b, pl.ds(m0, S_half), :].astype(jnp.bfloat16), wo,
                preferred_element_type=jnp.float32,
            )

        for c in range(N_CHUNKS):
            b, r = divmod(c, S_half // ROWS)
            rdmas[c].wait_recv()
            rows = pl.ds(r * ROWS, ROWS)
            out_ref[b, rows, :] = (
                out_ref[b, rows, :] + recv_buf[c].astype(jnp.float32)
            )
        for c in range(N_CHUNKS):
            rdmas[c].wait_send()

    out_shape = jax.ShapeDtypeStruct((B, S_half, N), jnp.float32)
    return pl.pallas_call(
        body,
        out_shape=out_shape,
        in_specs=[
            pl.BlockSpec(memory_space=pltpu.VMEM),
            pl.BlockSpec(memory_space=pltpu.VMEM),
        ],
        out_specs=pl.BlockSpec(memory_space=pltpu.VMEM),
        scratch_shapes=[
            pltpu.VMEM((N_CHUNKS, ROWS, N), jnp.bfloat16),
            pltpu.VMEM((N_CHUNKS, ROWS, N), jnp.bfloat16),
            pltpu.SemaphoreType.DMA((N_CHUNKS,)),
            pltpu.SemaphoreType.DMA((N_CHUNKS,)),
        ],
        compiler_params=pltpu.CompilerParams(collective_id=0),
    )(O2, Wo)
